# baseline (device time: 18647 ns/iter reference)
import jax
import jax.numpy as jnp
from jax import lax
from jax.experimental import pallas as pl
from jax.experimental.pallas import tpu as pltpu

N_Z = 4


def kernel(Q, K, V):
    b, kv, h, d = K.shape
    bh = b * h
    scale = d ** -0.5
    pk = d + 2

    Kt = K.transpose(0, 2, 3, 1).reshape(bh, d, kv)
    Vt = V.transpose(0, 2, 3, 1).reshape(bh, d, kv)
    Qt = Q.transpose(0, 2, 1, 3).reshape(bh, d)

    def body(q_ref, k_ref, v_ref, o_ref, loc_ref, comm_ref,
             send_sems, recv_sems):
        my_x = lax.axis_index("x")
        my_y = lax.axis_index("y")
        my_z = lax.axis_index("z")

        barrier_sem = pltpu.get_barrier_semaphore()
        for j in range(1, N_Z):
            pl.semaphore_signal(
                barrier_sem, inc=1,
                device_id=(my_x, my_y, (my_z + j) % N_Z),
                device_id_type=pl.DeviceIdType.MESH,
            )

        s = jnp.sum(k_ref[...] * q_ref[...][:, :, None], axis=1) * scale
        m_loc = jnp.max(s, axis=-1, keepdims=True)
        p = jnp.exp(s - m_loc)
        l_loc = jnp.sum(p, axis=-1, keepdims=True)
        o_loc = jnp.sum(v_ref[...] * p[:, None, :], axis=2)

        loc_ref[...] = jnp.concatenate([o_loc, m_loc, l_loc], axis=1)

        pl.semaphore_wait(barrier_sem, N_Z - 1)

        sends = []
        for j in range(1, N_Z):
            rdma = pltpu.make_async_remote_copy(
                src_ref=loc_ref,
                dst_ref=comm_ref.at[j - 1],
                send_sem=send_sems.at[j - 1],
                recv_sem=recv_sems.at[j - 1],
                device_id=(my_x, my_y, (my_z + j) % N_Z),
                device_id_type=pl.DeviceIdType.MESH,
            )
            rdma.start()
            sends.append(rdma)

        for j in range(1, N_Z):
            pltpu.make_async_remote_copy(
                src_ref=loc_ref,
                dst_ref=comm_ref.at[j - 1],
                send_sem=send_sems.at[j - 1],
                recv_sem=recv_sems.at[j - 1],
                device_id=(my_x, my_y, (my_z + j) % N_Z),
                device_id_type=pl.DeviceIdType.MESH,
            ).wait_recv()

        m_max = m_loc
        for jj in range(N_Z - 1):
            m_max = jnp.maximum(m_max, comm_ref[jj, :, d:d + 1])
        sc = jnp.exp(m_loc - m_max)
        num = o_loc * sc
        den = l_loc * sc
        for jj in range(N_Z - 1):
            sc = jnp.exp(comm_ref[jj, :, d:d + 1] - m_max)
            num = num + comm_ref[jj, :, 0:d] * sc
            den = den + comm_ref[jj, :, d + 1:d + 2] * sc
        o_ref[...] = num / den

        for rdma in sends:
            rdma.wait_send()

    out = pl.pallas_call(
        body,
        out_shape=jax.ShapeDtypeStruct((bh, d), jnp.float32),
        in_specs=[
            pl.BlockSpec(memory_space=pltpu.VMEM),
            pl.BlockSpec(memory_space=pltpu.VMEM),
            pl.BlockSpec(memory_space=pltpu.VMEM),
        ],
        out_specs=pl.BlockSpec(memory_space=pltpu.VMEM),
        scratch_shapes=[
            pltpu.VMEM((bh, pk), jnp.float32),
            pltpu.VMEM((N_Z - 1, bh, pk), jnp.float32),
            pltpu.SemaphoreType.DMA((N_Z - 1,)),
            pltpu.SemaphoreType.DMA((N_Z - 1,)),
        ],
        compiler_params=pltpu.CompilerParams(collective_id=0),
    )(Qt, Kt, Vt)

    return out.reshape(b, h, d)[:, None, :, :]
